# baseline (speedup 1.0000x reference)
import functools

import jax
import jax.numpy as jnp
from jax import lax
from jax.experimental import pallas as pl
from jax.experimental.pallas import tpu as pltpu
from jax.experimental.pallas import tpu_sc as plsc

S, N, K, D = 16, 16384, 256, 32

NB = 4096
NBLK = N // NB
HALF_S = S // 2
CHUNK = 128
WORKERS_PER_SUB = 2
PTS_PER_WORKER = N // WORKERS_PER_SUB
NCHUNK = PTS_PER_WORKER // CHUNK
ROWS_PER_TILE = HALF_S * K // 16
PADW = 128


def _assign_body(x_ref, c_ref, assigns_ref):
    x = x_ref[0]
    c = c_ref[0]
    c2 = jnp.sum(c * c, axis=1)
    xc = lax.dot_general(x, c * -2.0, (((1,), (1,)), ((), ())),
                         preferred_element_type=jnp.float32)
    dist = xc + c2[None, :]
    assigns_ref[0, 0, :] = jnp.argmin(dist, axis=1).astype(jnp.int32)


_assign_call = pl.pallas_call(
    _assign_body,
    grid=(S, NBLK),
    in_specs=[
        pl.BlockSpec((1, NB, D), lambda s, nb: (s, nb, 0)),
        pl.BlockSpec((1, K, D), lambda s, nb: (s, 0, 0)),
    ],
    out_specs=pl.BlockSpec((1, 1, NB), lambda s, nb: (s * NBLK + nb, 0, 0)),
    out_shape=jax.ShapeDtypeStruct((S * NBLK, 1, NB), jnp.int32),
)


def _scatter_body(x_hbm, assigns_hbm, cents_hbm, out_hbm,
                  xbuf0, xbuf1, xpad0, xpad1, ibuf0, ibuf1,
                  sem_x0, sem_x1, sem_i0, sem_i1, sem_s0, sem_s1, shared):
    cid = lax.axis_index("c")
    tid = lax.axis_index("s")
    sub = cid * HALF_S + (tid % HALF_S)
    half = tid // HALF_S
    row_off = (tid % HALF_S) * K
    zero16 = jnp.zeros((16,), jnp.float32)
    one16 = jnp.zeros((16,), jnp.float32) + 1.0
    xbufs, xpads, ibufs = (xbuf0, xbuf1), (xpad0, xpad1), (ibuf0, ibuf1)
    sems_x, sems_i, sems_s = (sem_x0, sem_x1), (sem_i0, sem_i1), (sem_s0, sem_s1)

    def zrow(r, carry):
        for cc in range(PADW // 16):
            xpad0[r, pl.ds(cc * 16, 16)] = zero16
        return carry

    lax.fori_loop(0, CHUNK, zrow, 0)
    pltpu.sync_copy(xpad0, shared.at[pl.ds(tid * ROWS_PER_TILE, ROWS_PER_TILE)])

    def onerow(r, carry):
        xpad0[r, pl.ds(D, 16)] = one16
        xpad1[r, pl.ds(D, 16)] = one16
        for cc in range(D // 16 + 1, PADW // 16):
            xpad1[r, pl.ds(cc * 16, 16)] = zero16
        return carry

    lax.fori_loop(0, CHUNK, onerow, 0)
    plsc.subcore_barrier()
    base_n = half * PTS_PER_WORKER

    def start_fetch(i, b):
        n0 = pl.multiple_of(base_n + i * CHUNK, CHUNK)
        n04 = pl.multiple_of(n0 // 4, CHUNK // 4)
        pltpu.make_async_copy(x_hbm.at[sub, pl.ds(n04, CHUNK // 4)], xbufs[b],
                              sems_x[b]).start()
        pltpu.make_async_copy(assigns_hbm.at[sub, pl.ds(n0, CHUNK)], ibufs[b],
                              sems_i[b]).start()

    def wait_fetch(i, b):
        n0 = pl.multiple_of(base_n + i * CHUNK, CHUNK)
        n04 = pl.multiple_of(n0 // 4, CHUNK // 4)
        pltpu.make_async_copy(x_hbm.at[sub, pl.ds(n04, CHUNK // 4)], xbufs[b],
                              sems_x[b]).wait()
        pltpu.make_async_copy(assigns_hbm.at[sub, pl.ds(n0, CHUNK)], ibufs[b],
                              sems_i[b]).wait()

    def do_chunk(i, b, drain_other):
        wait_fetch(i, b)

        def pad_row(r, carry):
            for q in range(4):
                xpads[b][r * 4 + q, pl.ds(0, 16)] = xbufs[b][r, pl.ds(q * 32, 16)]
                xpads[b][r * 4 + q, pl.ds(16, 16)] = xbufs[b][r, pl.ds(q * 32 + 16, 16)]
            return carry

        lax.fori_loop(0, CHUNK // 4, pad_row, 0)
        for j in range(CHUNK // 16):
            sl = pl.ds(j * 16, 16)
            ibufs[b][sl] = ibufs[b][sl] + row_off
        pltpu.make_async_copy(xpads[b], shared.at[ibufs[b]], sems_s[b]).start(add=True)
        @pl.when(i + 1 < NCHUNK)
        def _():
            if drain_other:
                pltpu.make_async_copy(
                    xpads[1 - b], shared.at[ibufs[1 - b]], sems_s[1 - b]).wait()
            start_fetch(i + 1, 1 - b)

    start_fetch(0, 0)
    do_chunk(0, 0, False)
    do_chunk(1, 1, True)

    def pair(g, carry):
        i = 2 + g * 2
        do_chunk(i, 0, True)
        do_chunk(i + 1, 1, True)
        return carry

    lax.fori_loop(0, (NCHUNK - 2) // 2, pair, 0)
    pltpu.make_async_copy(xpads[0], shared.at[ibufs[0]], sems_s[0]).wait()
    pltpu.make_async_copy(xpads[1], shared.at[ibufs[1]], sems_s[1]).wait()
    plsc.subcore_barrier()
    out_base = cid * (HALF_S * K)
    rows0 = pl.multiple_of((out_base + tid * ROWS_PER_TILE) // 4,
                           ROWS_PER_TILE // 4)
    pltpu.sync_copy(shared.at[pl.ds(tid * ROWS_PER_TILE, ROWS_PER_TILE)], xpad0)
    pltpu.sync_copy(cents_hbm.at[pl.ds(rows0, ROWS_PER_TILE // 4)], xbuf0)

    def mrow(r, carry):
        for q in range(4):
            tr = r * 4 + q
            cntv = xpad0[tr, pl.ds(D, 16)]
            denom = jnp.maximum(cntv, one16)
            keep = jnp.minimum(cntv, one16)
            drop = one16 - keep
            m0 = xpad0[tr, pl.ds(0, 16)] / denom
            m1 = xpad0[tr, pl.ds(16, 16)] / denom
            c0 = xbuf0[r, pl.ds(q * 32, 16)]
            c1 = xbuf0[r, pl.ds(q * 32 + 16, 16)]
            xbuf0[r, pl.ds(q * 32, 16)] = m0 * keep + c0 * drop
            xbuf0[r, pl.ds(q * 32 + 16, 16)] = m1 * keep + c1 * drop
        return carry

    lax.fori_loop(0, ROWS_PER_TILE // 4, mrow, 0)
    pltpu.sync_copy(xbuf0, out_hbm.at[pl.ds(rows0, ROWS_PER_TILE // 4)])


@functools.lru_cache(maxsize=1)
def _get_scatter_kernel():
    mesh = plsc.VectorSubcoreMesh(core_axis_name="c", subcore_axis_name="s")
    return pl.kernel(
        _scatter_body,
        mesh=mesh,
        out_type=jax.ShapeDtypeStruct((S * K // 4, PADW), jnp.float32),
        scratch_types=[
            pltpu.VMEM((CHUNK // 4, PADW), jnp.float32),
            pltpu.VMEM((CHUNK // 4, PADW), jnp.float32),
            pltpu.VMEM((CHUNK, PADW), jnp.float32),
            pltpu.VMEM((CHUNK, PADW), jnp.float32),
            pltpu.VMEM((CHUNK,), jnp.int32),
            pltpu.VMEM((CHUNK,), jnp.int32),
            pltpu.SemaphoreType.DMA,
            pltpu.SemaphoreType.DMA,
            pltpu.SemaphoreType.DMA,
            pltpu.SemaphoreType.DMA,
            pltpu.SemaphoreType.DMA,
            pltpu.SemaphoreType.DMA,
            pltpu.VMEM_SHARED((HALF_S * K, PADW), jnp.float32),
        ],
    )


def kernel(x, centroids):
    assigns3 = _assign_call(x, centroids)
    assigns = assigns3.reshape(S, N)
    x4 = x.reshape(S, N // 4, PADW)
    cents4 = centroids.reshape(S * K // 4, PADW)
    new_centroids = _get_scatter_kernel()(x4, assigns, cents4)
    return new_centroids.reshape(S, K, D), assigns


# speedup vs baseline: 1.1878x; 1.1878x over previous
import functools

import jax
import jax.numpy as jnp
from jax import lax
from jax.experimental import pallas as pl
from jax.experimental.pallas import tpu as pltpu
from jax.experimental.pallas import tpu_sc as plsc

S, N, K, D = 16, 16384, 256, 32

NB = 4096
NBLK = N // NB
HALF_S = S // 2
CHUNK = 128
WORKERS_PER_SUB = 2
PTS_PER_WORKER = N // WORKERS_PER_SUB
NCHUNK = PTS_PER_WORKER // CHUNK
ROWS_PER_TILE = HALF_S * K // 16
PADW = 128


def _assign_body(x_ref, c_ref, assigns_ref):
    x = x_ref[0]
    c = c_ref[0]
    c2 = jnp.sum(c * c, axis=1)
    xc = lax.dot_general(x, c * -2.0, (((1,), (1,)), ((), ())),
                         preferred_element_type=jnp.float32)
    dist = xc + c2[None, :]
    assigns_ref[0, 0, :] = jnp.argmin(dist, axis=1).astype(jnp.int32)


_assign_call = pl.pallas_call(
    _assign_body,
    grid=(S, NBLK),
    in_specs=[
        pl.BlockSpec((1, NB, D), lambda s, nb: (s, nb, 0)),
        pl.BlockSpec((1, K, D), lambda s, nb: (s, 0, 0)),
    ],
    out_specs=pl.BlockSpec((1, 1, NB), lambda s, nb: (s * NBLK + nb, 0, 0)),
    out_shape=jax.ShapeDtypeStruct((S * NBLK, 1, NB), jnp.int32),
)


def _scatter_body(x_hbm, assigns_hbm, cents_hbm, out_hbm,
                  xbuf0, xbuf1, xpad0, xpad1, ibuf0, ibuf1,
                  sem_x0, sem_x1, sem_i0, sem_i1, sem_s0, sem_s1, shared):
    cid = lax.axis_index("c")
    tid = lax.axis_index("s")
    sub = cid * HALF_S + (tid % HALF_S)
    half = tid // HALF_S
    row_off = (tid % HALF_S) * K
    zero16 = jnp.zeros((16,), jnp.float32)
    one16 = jnp.zeros((16,), jnp.float32) + 1.0
    xbufs, xpads, ibufs = (xbuf0, xbuf1), (xpad0, xpad1), (ibuf0, ibuf1)
    sems_x, sems_i, sems_s = (sem_x0, sem_x1), (sem_i0, sem_i1), (sem_s0, sem_s1)

    def zrow(r, carry):
        for cc in range(PADW // 16):
            xpad0[r, pl.ds(cc * 16, 16)] = zero16
        return carry

    lax.fori_loop(0, CHUNK, zrow, 0)
    pltpu.sync_copy(xpad0, shared.at[pl.ds(tid * ROWS_PER_TILE, ROWS_PER_TILE)])

    def onerow(r, carry):
        xpad0[r, pl.ds(D, 16)] = one16
        xpad1[r, pl.ds(D, 16)] = one16
        for cc in range(D // 16 + 1, PADW // 16):
            xpad1[r, pl.ds(cc * 16, 16)] = zero16
        return carry

    lax.fori_loop(0, CHUNK, onerow, 0)
    plsc.subcore_barrier()
    base_n = half * PTS_PER_WORKER

    def start_fetch(i, b):
        n0 = pl.multiple_of(base_n + i * CHUNK, CHUNK)
        pltpu.make_async_copy(x_hbm.at[sub, pl.ds(n0, CHUNK)], xbufs[b],
                              sems_x[b]).start()
        pltpu.make_async_copy(assigns_hbm.at[sub, pl.ds(n0, CHUNK)], ibufs[b],
                              sems_i[b]).start()

    def wait_fetch(i, b):
        n0 = pl.multiple_of(base_n + i * CHUNK, CHUNK)
        pltpu.make_async_copy(x_hbm.at[sub, pl.ds(n0, CHUNK)], xbufs[b],
                              sems_x[b]).wait()
        pltpu.make_async_copy(assigns_hbm.at[sub, pl.ds(n0, CHUNK)], ibufs[b],
                              sems_i[b]).wait()

    def do_chunk(i, b, drain_other):
        wait_fetch(i, b)

        def pad_row(r, carry):
            xpads[b][r, pl.ds(0, 16)] = xbufs[b][r, pl.ds(0, 16)]
            xpads[b][r, pl.ds(16, 16)] = xbufs[b][r, pl.ds(16, 16)]
            return carry

        lax.fori_loop(0, CHUNK, pad_row, 0)
        for j in range(CHUNK // 16):
            sl = pl.ds(j * 16, 16)
            ibufs[b][sl] = ibufs[b][sl] + row_off
        pltpu.make_async_copy(xpads[b], shared.at[ibufs[b]], sems_s[b]).start(add=True)
        @pl.when(i + 1 < NCHUNK)
        def _():
            if drain_other:
                pltpu.make_async_copy(
                    xpads[1 - b], shared.at[ibufs[1 - b]], sems_s[1 - b]).wait()
            start_fetch(i + 1, 1 - b)

    start_fetch(0, 0)
    do_chunk(0, 0, False)
    do_chunk(1, 1, True)

    def pair(g, carry):
        i = 2 + g * 2
        do_chunk(i, 0, True)
        do_chunk(i + 1, 1, True)
        return carry

    lax.fori_loop(0, (NCHUNK - 2) // 2, pair, 0)
    pltpu.make_async_copy(xpads[0], shared.at[ibufs[0]], sems_s[0]).wait()
    pltpu.make_async_copy(xpads[1], shared.at[ibufs[1]], sems_s[1]).wait()
    plsc.subcore_barrier()
    out_base = cid * (HALF_S * K)
    rows0 = pl.multiple_of(out_base + tid * ROWS_PER_TILE, ROWS_PER_TILE)
    pltpu.sync_copy(shared.at[pl.ds(tid * ROWS_PER_TILE, ROWS_PER_TILE)], xpad0)
    pltpu.sync_copy(cents_hbm.at[pl.ds(rows0, ROWS_PER_TILE)], xbuf0)

    def mrow(r, carry):
        cntv = xpad0[r, pl.ds(D, 16)]
        denom = jnp.maximum(cntv, one16)
        keep = jnp.minimum(cntv, one16)
        drop = one16 - keep
        m0 = xpad0[r, pl.ds(0, 16)] / denom
        m1 = xpad0[r, pl.ds(16, 16)] / denom
        xbuf0[r, pl.ds(0, 16)] = m0 * keep + xbuf0[r, pl.ds(0, 16)] * drop
        xbuf0[r, pl.ds(16, 16)] = m1 * keep + xbuf0[r, pl.ds(16, 16)] * drop
        return carry

    lax.fori_loop(0, ROWS_PER_TILE, mrow, 0)
    pltpu.sync_copy(xbuf0, out_hbm.at[pl.ds(rows0, ROWS_PER_TILE)])


@functools.lru_cache(maxsize=1)
def _get_scatter_kernel():
    mesh = plsc.VectorSubcoreMesh(core_axis_name="c", subcore_axis_name="s")
    return pl.kernel(
        _scatter_body,
        mesh=mesh,
        out_type=jax.ShapeDtypeStruct((S * K, D), jnp.float32),
        scratch_types=[
            pltpu.VMEM((CHUNK, D), jnp.float32),
            pltpu.VMEM((CHUNK, D), jnp.float32),
            pltpu.VMEM((CHUNK, PADW), jnp.float32),
            pltpu.VMEM((CHUNK, PADW), jnp.float32),
            pltpu.VMEM((CHUNK,), jnp.int32),
            pltpu.VMEM((CHUNK,), jnp.int32),
            pltpu.SemaphoreType.DMA,
            pltpu.SemaphoreType.DMA,
            pltpu.SemaphoreType.DMA,
            pltpu.SemaphoreType.DMA,
            pltpu.SemaphoreType.DMA,
            pltpu.SemaphoreType.DMA,
            pltpu.VMEM_SHARED((HALF_S * K, PADW), jnp.float32),
        ],
    )


def kernel(x, centroids):
    assigns3 = _assign_call(x, centroids)
    assigns = assigns3.reshape(S, N)
    new_centroids = _get_scatter_kernel()(x, assigns, centroids.reshape(S * K, D))
    return new_centroids.reshape(S, K, D), assigns


# speedup vs baseline: 1.2252x; 1.0315x over previous
import functools

import jax
import jax.numpy as jnp
from jax import lax
from jax.experimental import pallas as pl
from jax.experimental.pallas import tpu as pltpu
from jax.experimental.pallas import tpu_sc as plsc

S, N, K, D = 16, 16384, 256, 32

NB = 8192
NBLK = N // NB
HALF_S = S // 2
CHUNK = 128
WORKERS_PER_SUB = 2
PTS_PER_WORKER = N // WORKERS_PER_SUB
NCHUNK = PTS_PER_WORKER // CHUNK
ROWS_PER_TILE = HALF_S * K // 16
PADW = 128


def _assign_body(x_ref, c_ref, assigns_ref):
    x = x_ref[0]
    c = c_ref[0]
    c2 = jnp.sum(c * c, axis=1)
    xc = lax.dot_general(x, c * -2.0, (((1,), (1,)), ((), ())),
                         preferred_element_type=jnp.float32)
    dist = xc + c2[None, :]
    assigns_ref[0, 0, :] = jnp.argmin(dist, axis=1).astype(jnp.int32)


_assign_call = pl.pallas_call(
    _assign_body,
    grid=(S, NBLK),
    in_specs=[
        pl.BlockSpec((1, NB, D), lambda s, nb: (s, nb, 0)),
        pl.BlockSpec((1, K, D), lambda s, nb: (s, 0, 0)),
    ],
    out_specs=pl.BlockSpec((1, 1, NB), lambda s, nb: (s * NBLK + nb, 0, 0)),
    out_shape=jax.ShapeDtypeStruct((S * NBLK, 1, NB), jnp.int32),
)


def _scatter_body(x_hbm, assigns_hbm, cents_hbm, out_hbm,
                  xbuf0, xbuf1, xpad0, xpad1, ibuf0, ibuf1,
                  sem_x0, sem_x1, sem_i0, sem_i1, sem_s0, sem_s1, shared):
    cid = lax.axis_index("c")
    tid = lax.axis_index("s")
    sub = cid * HALF_S + (tid % HALF_S)
    half = tid // HALF_S
    row_off = (tid % HALF_S) * K
    zero16 = jnp.zeros((16,), jnp.float32)
    one16 = jnp.zeros((16,), jnp.float32) + 1.0
    xbufs, xpads, ibufs = (xbuf0, xbuf1), (xpad0, xpad1), (ibuf0, ibuf1)
    sems_x, sems_i, sems_s = (sem_x0, sem_x1), (sem_i0, sem_i1), (sem_s0, sem_s1)

    def zrow(r, carry):
        for cc in range(PADW // 16):
            xpad0[r, pl.ds(cc * 16, 16)] = zero16
        return carry

    lax.fori_loop(0, CHUNK, zrow, 0)
    pltpu.sync_copy(xpad0, shared.at[pl.ds(tid * ROWS_PER_TILE, ROWS_PER_TILE)])

    def onerow(r, carry):
        xpad0[r, pl.ds(D, 16)] = one16
        xpad1[r, pl.ds(D, 16)] = one16
        for cc in range(D // 16 + 1, PADW // 16):
            xpad1[r, pl.ds(cc * 16, 16)] = zero16
        return carry

    lax.fori_loop(0, CHUNK, onerow, 0)
    plsc.subcore_barrier()
    base_n = half * PTS_PER_WORKER

    def start_fetch(i, b):
        n0 = pl.multiple_of(base_n + i * CHUNK, CHUNK)
        pltpu.make_async_copy(x_hbm.at[sub, pl.ds(n0, CHUNK)], xbufs[b],
                              sems_x[b]).start()
        pltpu.make_async_copy(assigns_hbm.at[sub, pl.ds(n0, CHUNK)], ibufs[b],
                              sems_i[b]).start()

    def wait_fetch(i, b):
        n0 = pl.multiple_of(base_n + i * CHUNK, CHUNK)
        pltpu.make_async_copy(x_hbm.at[sub, pl.ds(n0, CHUNK)], xbufs[b],
                              sems_x[b]).wait()
        pltpu.make_async_copy(assigns_hbm.at[sub, pl.ds(n0, CHUNK)], ibufs[b],
                              sems_i[b]).wait()

    def do_chunk(i, b, drain_other):
        wait_fetch(i, b)

        def pad_row(r, carry):
            xpads[b][r, pl.ds(0, 16)] = xbufs[b][r, pl.ds(0, 16)]
            xpads[b][r, pl.ds(16, 16)] = xbufs[b][r, pl.ds(16, 16)]
            return carry

        lax.fori_loop(0, CHUNK, pad_row, 0)
        for j in range(CHUNK // 16):
            sl = pl.ds(j * 16, 16)
            ibufs[b][sl] = ibufs[b][sl] + row_off
        pltpu.make_async_copy(xpads[b], shared.at[ibufs[b]], sems_s[b]).start(add=True)
        @pl.when(i + 1 < NCHUNK)
        def _():
            if drain_other:
                pltpu.make_async_copy(
                    xpads[1 - b], shared.at[ibufs[1 - b]], sems_s[1 - b]).wait()
            start_fetch(i + 1, 1 - b)

    start_fetch(0, 0)
    do_chunk(0, 0, False)
    do_chunk(1, 1, True)

    def pair(g, carry):
        i = 2 + g * 2
        do_chunk(i, 0, True)
        do_chunk(i + 1, 1, True)
        return carry

    lax.fori_loop(0, (NCHUNK - 2) // 2, pair, 0)
    pltpu.make_async_copy(xpads[0], shared.at[ibufs[0]], sems_s[0]).wait()
    pltpu.make_async_copy(xpads[1], shared.at[ibufs[1]], sems_s[1]).wait()
    plsc.subcore_barrier()
    out_base = cid * (HALF_S * K)
    rows0 = pl.multiple_of(out_base + tid * ROWS_PER_TILE, ROWS_PER_TILE)
    pltpu.sync_copy(shared.at[pl.ds(tid * ROWS_PER_TILE, ROWS_PER_TILE)], xpad0)
    pltpu.sync_copy(cents_hbm.at[pl.ds(rows0, ROWS_PER_TILE)], xbuf0)

    def mrow(r, carry):
        cntv = xpad0[r, pl.ds(D, 16)]
        denom = jnp.maximum(cntv, one16)
        keep = jnp.minimum(cntv, one16)
        drop = one16 - keep
        m0 = xpad0[r, pl.ds(0, 16)] / denom
        m1 = xpad0[r, pl.ds(16, 16)] / denom
        xbuf0[r, pl.ds(0, 16)] = m0 * keep + xbuf0[r, pl.ds(0, 16)] * drop
        xbuf0[r, pl.ds(16, 16)] = m1 * keep + xbuf0[r, pl.ds(16, 16)] * drop
        return carry

    lax.fori_loop(0, ROWS_PER_TILE, mrow, 0)
    pltpu.sync_copy(xbuf0, out_hbm.at[pl.ds(rows0, ROWS_PER_TILE)])


@functools.lru_cache(maxsize=1)
def _get_scatter_kernel():
    mesh = plsc.VectorSubcoreMesh(core_axis_name="c", subcore_axis_name="s")
    return pl.kernel(
        _scatter_body,
        mesh=mesh,
        out_type=jax.ShapeDtypeStruct((S * K, D), jnp.float32),
        scratch_types=[
            pltpu.VMEM((CHUNK, D), jnp.float32),
            pltpu.VMEM((CHUNK, D), jnp.float32),
            pltpu.VMEM((CHUNK, PADW), jnp.float32),
            pltpu.VMEM((CHUNK, PADW), jnp.float32),
            pltpu.VMEM((CHUNK,), jnp.int32),
            pltpu.VMEM((CHUNK,), jnp.int32),
            pltpu.SemaphoreType.DMA,
            pltpu.SemaphoreType.DMA,
            pltpu.SemaphoreType.DMA,
            pltpu.SemaphoreType.DMA,
            pltpu.SemaphoreType.DMA,
            pltpu.SemaphoreType.DMA,
            pltpu.VMEM_SHARED((HALF_S * K, PADW), jnp.float32),
        ],
    )


def kernel(x, centroids):
    assigns3 = _assign_call(x, centroids)
    assigns = assigns3.reshape(S, N)
    new_centroids = _get_scatter_kernel()(x, assigns, centroids.reshape(S * K, D))
    return new_centroids.reshape(S, K, D), assigns
